# baseline (speedup 1.0000x reference)
import functools

import jax
import jax.numpy as jnp
from jax import lax
from jax.experimental import pallas as pl
from jax.experimental.pallas import tpu as pltpu
from jax.experimental.pallas import tpu_sc as plsc

B, L = 4096, 200
VOCAB, D, HIDDEN = 100000, 128, 64

_NC, _NS = 2, 16
_NW = _NC * _NS
_RPW = B // _NW
_NCHUNK = 2
_CH = L // _NCHUNK
_DV = D // 16


def _sc_gather_sum(x3, tok_emb):
    mesh = plsc.VectorSubcoreMesh(core_axis_name="c", subcore_axis_name="s")

    @functools.partial(
        pl.kernel,
        mesh=mesh,
        out_type=jax.ShapeDtypeStruct((B, D), jnp.float32),
        scratch_types=[
            pltpu.VMEM((_RPW, _NCHUNK, _CH), jnp.int32),
            pltpu.VMEM((_CH, D), jnp.float32),
            pltpu.VMEM((_CH, D), jnp.float32),
            pltpu.VMEM((_CH, D), jnp.float32),
            pltpu.VMEM((_CH, D), jnp.float32),
            pltpu.VMEM((_CH, D), jnp.float32),
            pltpu.VMEM((_CH, D), jnp.float32),
            pltpu.VMEM((_RPW, D), jnp.float32),
            pltpu.SemaphoreType.DMA,
            pltpu.SemaphoreType.DMA,
            pltpu.SemaphoreType.DMA,
            pltpu.SemaphoreType.DMA,
            pltpu.SemaphoreType.DMA,
            pltpu.SemaphoreType.DMA,
        ],
    )
    def k(x_hbm, tab_hbm, out_hbm, idx_v,
          buf0, buf1, buf2, buf3, buf4, buf5, out_v,
          sem0, sem1, sem2, sem3, sem4, sem5):
        wid = lax.axis_index("s") * _NC + lax.axis_index("c")
        base = wid * _RPW
        pltpu.sync_copy(x_hbm.at[pl.ds(base, _RPW)], idx_v)

        bufs = (buf0, buf1, buf2, buf3, buf4, buf5)
        sems = (sem0, sem1, sem2, sem3, sem4, sem5)

        def issue(r, c, buf, sem):
            pltpu.async_copy(tab_hbm.at[idx_v.at[r, c]], buf, sem)

        def drain(r, c, buf, sem):
            pltpu.make_async_copy(tab_hbm.at[idx_v.at[r, c]], buf, sem).wait()

        def accum_chunk(buf, accs):
            @plsc.parallel_loop(0, _CH // 2, unroll=4, carry=accs)
            def out(l, accs):
                res = []
                for j in range(_DV):
                    a = accs[j] + buf[2 * l, pl.ds(j * 16, 16)]
                    res.append(a + buf[2 * l + 1, pl.ds(j * 16, 16)])
                return tuple(res)
            return out

        for m in range(6):
            issue(m // 2, m % 2, bufs[m], sems[m])

        zero = tuple(jnp.zeros((16,), jnp.float32) for _ in range(_DV))

        def process_row(r, half, do_issue):
            accs = zero
            for c in range(_NCHUNK):
                m = 2 * half + c
                drain(r, c, bufs[m], sems[m])
                accs = accum_chunk(bufs[m], accs)
                if do_issue:
                    @pl.when(r + 3 < _RPW)
                    def _():
                        issue(r + 3, c, bufs[m], sems[m])
            for j in range(_DV):
                out_v[r, pl.ds(j * 16, 16)] = accs[j]

        _K = _RPW // 3

        def body(k3, carry):
            r0 = 3 * k3
            for half in range(3):
                process_row(r0 + half, half, True)
            return carry

        lax.fori_loop(0, _K, body, 0)
        process_row(3 * _K, 0, False)
        process_row(3 * _K + 1, 1, False)
        pltpu.sync_copy(out_v, out_hbm.at[pl.ds(base, _RPW)])

    return k(x3, tok_emb)


def _tc_mask_pos(x, pos_emb):
    def body(x_ref, pos_ref, pm_ref, cnt_ref):
        valid = (x_ref[...] != 0).astype(jnp.float32)
        cnt_ref[...] = jnp.sum(valid, axis=1, keepdims=True)
        pm_ref[...] = jnp.dot(valid, pos_ref[...],
                              preferred_element_type=jnp.float32)

    return pl.pallas_call(
        body,
        out_shape=(jax.ShapeDtypeStruct((B, D), jnp.float32),
                   jax.ShapeDtypeStruct((B, 1), jnp.float32)),
    )(x, pos_emb)


def _tc_head(sc_sum, posmat, cnt, tok0, ln_g, ln_b, W1, b1, W2, b2):
    def body(s_ref, pm_ref, cnt_ref, t0_ref, g_ref, bb_ref,
             w1_ref, b1_ref, w2_ref, b2_ref, o_ref):
        cnt = cnt_ref[...]
        pooled = (
            s_ref[...]
            - (jnp.float32(L) - cnt) * t0_ref[...]
            + pm_ref[...]
        ) / jnp.maximum(cnt, 1.0)
        mu = jnp.mean(pooled, axis=1, keepdims=True)
        var = jnp.mean((pooled - mu) ** 2, axis=1, keepdims=True)
        hn = (pooled - mu) / jnp.sqrt(var + 1e-5) * g_ref[...] + bb_ref[...]
        z = jnp.dot(hn, w1_ref[...], preferred_element_type=jnp.float32) + b1_ref[...]
        z = 0.5 * z * (1.0 + lax.erf(z * jnp.float32(0.7071067811865476)))
        score = jnp.dot(z, w2_ref[...], preferred_element_type=jnp.float32) + b2_ref[...]
        o_ref[...] = score - jnp.mean(score)

    return pl.pallas_call(
        body,
        out_shape=jax.ShapeDtypeStruct((B, 1), jnp.float32),
    )(sc_sum, posmat, cnt, tok0, ln_g, ln_b, W1, b1, W2, b2)


def kernel(x, tok_emb, pos_emb, ln_g, ln_b, W1, b1, W2, b2):
    x3 = x.reshape(B, _NCHUNK, _CH)
    sc_sum = _sc_gather_sum(x3, tok_emb)
    posmat, cnt = _tc_mask_pos(x, pos_emb)
    score = _tc_head(
        sc_sum, posmat, cnt, tok_emb[0:1],
        ln_g.reshape(1, D), ln_b.reshape(1, D),
        W1, b1.reshape(1, HIDDEN), W2, b2.reshape(1, 1),
    )
    return score[:, 0]


# speedup vs baseline: 1.0136x; 1.0136x over previous
import functools

import jax
import jax.numpy as jnp
from jax import lax
from jax.experimental import pallas as pl
from jax.experimental.pallas import tpu as pltpu
from jax.experimental.pallas import tpu_sc as plsc

B, L = 4096, 200
VOCAB, D, HIDDEN = 100000, 128, 64

_NC, _NS = 2, 16
_NW = _NC * _NS
_RPW = B // _NW
_NCHUNK = 2
_CH = L // _NCHUNK
_DV = D // 16


def _sc_gather_sum(x3, tok_emb):
    mesh = plsc.VectorSubcoreMesh(core_axis_name="c", subcore_axis_name="s")

    @functools.partial(
        pl.kernel,
        mesh=mesh,
        out_type=jax.ShapeDtypeStruct((B, D), jnp.float32),
        scratch_types=[
            pltpu.VMEM((_RPW, _NCHUNK, _CH), jnp.int32),
            pltpu.VMEM((_CH, D), jnp.float32),
            pltpu.VMEM((_CH, D), jnp.float32),
            pltpu.VMEM((_CH, D), jnp.float32),
            pltpu.VMEM((_CH, D), jnp.float32),
            pltpu.VMEM((_CH, D), jnp.float32),
            pltpu.VMEM((_CH, D), jnp.float32),
            pltpu.VMEM((_RPW, D), jnp.float32),
            pltpu.SemaphoreType.DMA,
            pltpu.SemaphoreType.DMA,
            pltpu.SemaphoreType.DMA,
            pltpu.SemaphoreType.DMA,
            pltpu.SemaphoreType.DMA,
            pltpu.SemaphoreType.DMA,
        ],
    )
    def k(x_hbm, tab_hbm, out_hbm, idx_v,
          buf0, buf1, buf2, buf3, buf4, buf5, out_v,
          sem0, sem1, sem2, sem3, sem4, sem5):
        wid = lax.axis_index("s") * _NC + lax.axis_index("c")
        base = wid * _RPW
        pltpu.sync_copy(x_hbm.at[pl.ds(base, _RPW)], idx_v)

        bufs = (buf0, buf1, buf2, buf3, buf4, buf5)
        sems = (sem0, sem1, sem2, sem3, sem4, sem5)

        def issue(r, c, buf, sem):
            pltpu.async_copy(tab_hbm.at[idx_v.at[r, c]], buf, sem)

        def drain(r, c, buf, sem):
            pltpu.make_async_copy(tab_hbm.at[idx_v.at[r, c]], buf, sem).wait()

        def accum_chunk(buf, accs):
            @plsc.parallel_loop(0, _CH // 2, unroll=4, carry=accs)
            def out(l, accs):
                res = []
                for j in range(_DV):
                    a = accs[j] + buf[2 * l, pl.ds(j * 16, 16)]
                    res.append(a + buf[2 * l + 1, pl.ds(j * 16, 16)])
                return tuple(res)
            return out

        for m in range(6):
            issue(m // 2, m % 2, bufs[m], sems[m])

        zero = tuple(jnp.zeros((16,), jnp.float32) for _ in range(_DV))

        def process_row(r, half, do_issue):
            accs = zero
            for c in range(_NCHUNK):
                m = 2 * half + c
                drain(r, c, bufs[m], sems[m])
                accs = accum_chunk(bufs[m], accs)
                if do_issue:
                    @pl.when(r + 3 < _RPW)
                    def _():
                        issue(r + 3, c, bufs[m], sems[m])
            for j in range(_DV):
                out_v[r, pl.ds(j * 16, 16)] = accs[j]

        _K = _RPW // 3

        def body(k3, carry):
            r0 = 3 * k3
            for half in range(3):
                process_row(r0 + half, half, True)
            return carry

        lax.fori_loop(0, _K, body, 0)
        process_row(3 * _K, 0, False)
        process_row(3 * _K + 1, 1, False)
        pltpu.sync_copy(out_v, out_hbm.at[pl.ds(base, _RPW)])

    return k(x3, tok_emb)


def _tc_head(x, sc_sum, tok0, pos_emb, ln_g, ln_b, W1, b1, W2, b2):
    def body(x_ref, s_ref, t0_ref, pos_ref, g_ref, bb_ref,
             w1_ref, b1_ref, w2_ref, b2_ref, o_ref):
        valid = (x_ref[...] != 0).astype(jnp.float32)
        cnt = jnp.sum(valid, axis=1, keepdims=True)
        pooled = (
            s_ref[...]
            - (jnp.float32(L) - cnt) * t0_ref[...]
            + jnp.dot(valid, pos_ref[...], preferred_element_type=jnp.float32)
        ) / jnp.maximum(cnt, 1.0)
        mu = jnp.mean(pooled, axis=1, keepdims=True)
        var = jnp.mean((pooled - mu) ** 2, axis=1, keepdims=True)
        hn = (pooled - mu) / jnp.sqrt(var + 1e-5) * g_ref[...] + bb_ref[...]
        z = jnp.dot(hn, w1_ref[...], preferred_element_type=jnp.float32) + b1_ref[...]
        z = 0.5 * z * (1.0 + lax.erf(z * jnp.float32(0.7071067811865476)))
        score = jnp.dot(z, w2_ref[...], preferred_element_type=jnp.float32) + b2_ref[...]
        o_ref[...] = score - jnp.mean(score)

    return pl.pallas_call(
        body,
        out_shape=jax.ShapeDtypeStruct((B, 1), jnp.float32),
    )(x, sc_sum, tok0, pos_emb, ln_g, ln_b, W1, b1, W2, b2)


def kernel(x, tok_emb, pos_emb, ln_g, ln_b, W1, b1, W2, b2):
    x3 = x.reshape(B, _NCHUNK, _CH)
    sc_sum = _sc_gather_sum(x3, tok_emb)
    score = _tc_head(
        x, sc_sum, tok_emb[0:1], pos_emb,
        ln_g.reshape(1, D), ln_b.reshape(1, D),
        W1, b1.reshape(1, HIDDEN), W2, b2.reshape(1, 1),
    )
    return score[:, 0]
